# baseline (speedup 1.0000x reference)
import functools

import jax
import jax.numpy as jnp
from jax import lax
from jax.experimental import pallas as pl
from jax.experimental.pallas import tpu as pltpu

N_CHUNKS = 160000
N_CELLS = 10000
IN_DIM = 256
HID = 512
OUT_DIM = 256

T = 512
W = 128
NT = (N_CHUNKS + T - 1) // T
PAD_CHUNKS = NT * T
PAD_ID = N_CELLS
PAD_CELLS = 10240
CT = 1000


def _phi_seg_body(ids_smem, x_ref, ids_vec_ref, w0_ref, b0_ref, w1_ref,
                  b1_ref, agg_ref):
    i = pl.program_id(0)

    @pl.when(i == 0)
    def _init():
        agg_ref[...] = jnp.zeros_like(agg_ref)

    x = x_ref[...]
    h = jnp.dot(x, w0_ref[...], preferred_element_type=jnp.float32)
    h = jnp.maximum(h + b0_ref[...], 0.0)
    h = jnp.dot(h.astype(jnp.bfloat16), w1_ref[...],
                preferred_element_type=jnp.float32)
    h = jnp.maximum(h + b1_ref[...], 0.0)

    ids_vec = ids_vec_ref[0]

    def cond(p):
        return p < T

    def body(p):
        base = ids_smem[0, 0, p]
        base8 = (base // 8) * 8
        local = ids_vec - base8
        rows = lax.broadcasted_iota(jnp.int32, (W, T), 0)
        onehot = (rows == local).astype(jnp.bfloat16)
        h_hi = h.astype(jnp.bfloat16)
        h_lo = (h - h_hi.astype(jnp.float32)).astype(jnp.bfloat16)
        partial = (jnp.dot(onehot, h_hi, preferred_element_type=jnp.float32)
                   + jnp.dot(onehot, h_lo, preferred_element_type=jnp.float32))
        agg_ref[pl.ds(base8, W), :] += partial
        p_new = jnp.sum((ids_vec < base8 + W).astype(jnp.int32))
        return p_new

    lax.while_loop(cond, body, jnp.int32(0))


def _rho_body(a_ref, w0_ref, b0_ref, w1_ref, b1_ref, w2_ref, b2_ref, o_ref):
    a = a_ref[...]
    a_hi = a.astype(jnp.bfloat16)
    a_lo = (a - a_hi.astype(jnp.float32)).astype(jnp.bfloat16)
    r = (jnp.dot(a_hi, w0_ref[...], preferred_element_type=jnp.float32)
         + jnp.dot(a_lo, w0_ref[...], preferred_element_type=jnp.float32))
    r = jnp.maximum(r + b0_ref[...], 0.0)
    r = jnp.dot(r.astype(jnp.bfloat16), w1_ref[...],
                preferred_element_type=jnp.float32)
    r = jnp.maximum(r + b1_ref[...], 0.0)
    o_ref[...] = jnp.dot(r.astype(jnp.bfloat16), w2_ref[...],
                         preferred_element_type=jnp.float32) + b2_ref[...]


def kernel(chunk_features, segment_ids, phi_w0, phi_b0, phi_w1, phi_b1,
           rho_w0, rho_b0, rho_w1, rho_b1, rho_w2, rho_b2):
    ids = segment_ids.astype(jnp.int32)
    pad = PAD_CHUNKS - N_CHUNKS
    x = jnp.concatenate(
        [chunk_features.astype(jnp.bfloat16),
         jnp.zeros((pad, IN_DIM), jnp.bfloat16)], axis=0)
    ids = jnp.concatenate([ids, jnp.full((pad,), PAD_ID, jnp.int32)])
    ids3 = ids.reshape(NT, 1, T)

    agg = pl.pallas_call(
        _phi_seg_body,
        grid=(NT,),
        in_specs=[
            pl.BlockSpec((1, 1, T), lambda i: (i, 0, 0),
                         memory_space=pltpu.SMEM),
            pl.BlockSpec((T, IN_DIM), lambda i: (i, 0)),
            pl.BlockSpec((1, 1, T), lambda i: (i, 0, 0)),
            pl.BlockSpec((IN_DIM, HID), lambda i: (0, 0)),
            pl.BlockSpec((1, HID), lambda i: (0, 0)),
            pl.BlockSpec((HID, HID), lambda i: (0, 0)),
            pl.BlockSpec((1, HID), lambda i: (0, 0)),
        ],
        out_specs=pl.BlockSpec((PAD_CELLS, HID), lambda i: (0, 0)),
        out_shape=jax.ShapeDtypeStruct((PAD_CELLS, HID), jnp.float32),
        compiler_params=pltpu.CompilerParams(
            dimension_semantics=("arbitrary",)),
    )(ids3, x, ids3, phi_w0.astype(jnp.bfloat16), phi_b0.reshape(1, HID),
      phi_w1.astype(jnp.bfloat16), phi_b1.reshape(1, HID))

    agg = agg[:N_CELLS]

    out = pl.pallas_call(
        _rho_body,
        grid=(N_CELLS // CT,),
        in_specs=[
            pl.BlockSpec((CT, HID), lambda i: (i, 0)),
            pl.BlockSpec((HID, HID), lambda i: (0, 0)),
            pl.BlockSpec((1, HID), lambda i: (0, 0)),
            pl.BlockSpec((HID, HID), lambda i: (0, 0)),
            pl.BlockSpec((1, HID), lambda i: (0, 0)),
            pl.BlockSpec((HID, OUT_DIM), lambda i: (0, 0)),
            pl.BlockSpec((1, OUT_DIM), lambda i: (0, 0)),
        ],
        out_specs=pl.BlockSpec((CT, OUT_DIM), lambda i: (i, 0)),
        out_shape=jax.ShapeDtypeStruct((N_CELLS, OUT_DIM), jnp.float32),
    )(agg, rho_w0.astype(jnp.bfloat16), rho_b0.reshape(1, HID),
      rho_w1.astype(jnp.bfloat16), rho_b1.reshape(1, HID),
      rho_w2.astype(jnp.bfloat16), rho_b2.reshape(1, OUT_DIM))
    return out


# speedup vs baseline: 1.8218x; 1.8218x over previous
import jax
import jax.numpy as jnp
from jax import lax
from jax.experimental import pallas as pl
from jax.experimental.pallas import tpu as pltpu

N_CHUNKS = 160000
N_CELLS = 10000
IN_DIM = 256
HID = 512
OUT_DIM = 256

T = 1024
W = 128
NT = (N_CHUNKS + T - 1) // T
PAD_CHUNKS = NT * T
PAD_ID = N_CELLS
PAD_CELLS = 10240
CT = 1000


def _phi_seg_body(ids_smem, x_ref, ids_vec_ref, w0_ref, b0_ref, w1_ref,
                  b1_ref, agg_ref, hb0, hb1):
    i = pl.program_id(0)

    @pl.when(i == 0)
    def _init():
        agg_ref[...] = jnp.zeros_like(agg_ref)

    def compute_phi(h_out):
        row0 = i * T
        valid = (lax.broadcasted_iota(jnp.int32, (T, 1), 0) + row0) < N_CHUNKS
        x = jnp.where(valid, x_ref[...], 0.0).astype(jnp.bfloat16)
        h = jnp.dot(x, w0_ref[...], preferred_element_type=jnp.float32)
        h = jnp.maximum(h + b0_ref[...], 0.0).astype(jnp.bfloat16)
        h = jnp.dot(h, w1_ref[...], preferred_element_type=jnp.float32)
        h_out[...] = jnp.maximum(h + b1_ref[...], 0.0).astype(jnp.bfloat16)

    def scatter(h_in):
        h = h_in[...]
        ids_vec = ids_vec_ref[0]

        def cond(p):
            return p < T

        def body(p):
            base = ids_smem[0, 0, p]
            base8 = (base // 8) * 8
            local = ids_vec - base8
            rows = lax.broadcasted_iota(jnp.int32, (W, T), 0)
            onehot = (rows == local).astype(jnp.bfloat16)
            partial = jnp.dot(onehot, h, preferred_element_type=jnp.float32)
            agg_ref[pl.ds(base8, W), :] += partial
            return jnp.sum((ids_vec < base8 + W).astype(jnp.int32))

        lax.while_loop(cond, body, jnp.int32(0))

    even = i % 2 == 0

    @pl.when(jnp.logical_and(i < NT, even))
    def _():
        compute_phi(hb0)

    @pl.when(jnp.logical_and(i < NT, jnp.logical_not(even)))
    def _():
        compute_phi(hb1)

    @pl.when(jnp.logical_and(i > 0, even))
    def _():
        scatter(hb1)

    @pl.when(jnp.logical_and(i > 0, jnp.logical_not(even)))
    def _():
        scatter(hb0)


def _rho_body(a_ref, w0_ref, b0_ref, w1_ref, b1_ref, w2_ref, b2_ref, o_ref):
    a = a_ref[...].astype(jnp.bfloat16)
    r = jnp.dot(a, w0_ref[...], preferred_element_type=jnp.float32)
    r = jnp.maximum(r + b0_ref[...], 0.0)
    r = jnp.dot(r.astype(jnp.bfloat16), w1_ref[...],
                preferred_element_type=jnp.float32)
    r = jnp.maximum(r + b1_ref[...], 0.0)
    o_ref[...] = jnp.dot(r.astype(jnp.bfloat16), w2_ref[...],
                         preferred_element_type=jnp.float32) + b2_ref[...]


def kernel(chunk_features, segment_ids, phi_w0, phi_b0, phi_w1, phi_b1,
           rho_w0, rho_b0, rho_w1, rho_b1, rho_w2, rho_b2):
    ids = segment_ids.astype(jnp.int32)
    pad = PAD_CHUNKS - N_CHUNKS
    ids = jnp.concatenate([ids, jnp.full((pad,), PAD_ID, jnp.int32)])
    ids3 = ids.reshape(NT, 1, T)

    prev = lambda i: (jnp.maximum(i - 1, 0), 0, 0)
    cur_x = lambda i: (jnp.minimum(i, NT - 1), 0)

    agg = pl.pallas_call(
        _phi_seg_body,
        grid=(NT + 1,),
        in_specs=[
            pl.BlockSpec((1, 1, T), prev, memory_space=pltpu.SMEM),
            pl.BlockSpec((T, IN_DIM), cur_x),
            pl.BlockSpec((1, 1, T), prev),
            pl.BlockSpec((IN_DIM, HID), lambda i: (0, 0)),
            pl.BlockSpec((1, HID), lambda i: (0, 0)),
            pl.BlockSpec((HID, HID), lambda i: (0, 0)),
            pl.BlockSpec((1, HID), lambda i: (0, 0)),
        ],
        out_specs=pl.BlockSpec((PAD_CELLS, HID), lambda i: (0, 0)),
        out_shape=jax.ShapeDtypeStruct((PAD_CELLS, HID), jnp.float32),
        scratch_shapes=[
            pltpu.VMEM((T, HID), jnp.bfloat16),
            pltpu.VMEM((T, HID), jnp.bfloat16),
        ],
        compiler_params=pltpu.CompilerParams(
            dimension_semantics=("arbitrary",)),
    )(ids3, chunk_features, ids3, phi_w0.astype(jnp.bfloat16),
      phi_b0.reshape(1, HID), phi_w1.astype(jnp.bfloat16),
      phi_b1.reshape(1, HID))

    out = pl.pallas_call(
        _rho_body,
        grid=(N_CELLS // CT,),
        in_specs=[
            pl.BlockSpec((CT, HID), lambda i: (i, 0)),
            pl.BlockSpec((HID, HID), lambda i: (0, 0)),
            pl.BlockSpec((1, HID), lambda i: (0, 0)),
            pl.BlockSpec((HID, HID), lambda i: (0, 0)),
            pl.BlockSpec((1, HID), lambda i: (0, 0)),
            pl.BlockSpec((HID, OUT_DIM), lambda i: (0, 0)),
            pl.BlockSpec((1, OUT_DIM), lambda i: (0, 0)),
        ],
        out_specs=pl.BlockSpec((CT, OUT_DIM), lambda i: (i, 0)),
        out_shape=jax.ShapeDtypeStruct((N_CELLS, OUT_DIM), jnp.float32),
    )(agg, rho_w0.astype(jnp.bfloat16), rho_b0.reshape(1, HID),
      rho_w1.astype(jnp.bfloat16), rho_b1.reshape(1, HID),
      rho_w2.astype(jnp.bfloat16), rho_b2.reshape(1, OUT_DIM))
    return out


# speedup vs baseline: 2.0689x; 1.1356x over previous
import functools

import jax
import jax.numpy as jnp
from jax import lax
from jax.experimental import pallas as pl
from jax.experimental.pallas import tpu as pltpu

N_CHUNKS = 160000
N_CELLS = 10000
IN_DIM = 256
HID = 512
OUT_DIM = 256

T = 2048
W = 256
NT = (N_CHUNKS + T - 1) // T
PAD_CHUNKS = NT * T
PAD_ID = N_CELLS
PAD_CELLS = 10496
CT = 1000


def _phi_seg_body(ids_smem, x_ref, ids_vec_ref, w0_ref, b0_ref, w1_ref,
                  b1_ref, agg_ref):
    i = pl.program_id(0)

    @pl.when(i == 0)
    def _init():
        agg_ref[...] = jnp.zeros_like(agg_ref)

    row0 = i * T
    valid = (lax.broadcasted_iota(jnp.int32, (T, 1), 0) + row0) < N_CHUNKS
    x = jnp.where(valid, x_ref[...], 0.0).astype(jnp.bfloat16)
    h = jnp.dot(x, w0_ref[...], preferred_element_type=jnp.float32)
    h = jnp.maximum(h + b0_ref[...], 0.0).astype(jnp.bfloat16)
    h = jnp.dot(h, w1_ref[...], preferred_element_type=jnp.float32)
    h = jnp.maximum(h + b1_ref[...], 0.0).astype(jnp.bfloat16)

    ids_vec = ids_vec_ref[0]

    def cond(p):
        return p < T

    def body(p):
        base = ids_smem[0, 0, p]
        base8 = (base // 8) * 8
        local = ids_vec - base8
        rows = lax.broadcasted_iota(jnp.int32, (W, T), 0)
        onehot = (rows == local).astype(jnp.bfloat16)
        partial = jnp.dot(onehot, h, preferred_element_type=jnp.float32)
        agg_ref[pl.ds(base8, W), :] += partial
        p_new = jnp.sum((ids_vec < base8 + W).astype(jnp.int32))
        return p_new

    lax.while_loop(cond, body, jnp.int32(0))


def _rho_body(a_ref, w0_ref, b0_ref, w1_ref, b1_ref, w2_ref, b2_ref, o_ref):
    a = a_ref[...].astype(jnp.bfloat16)
    r = jnp.dot(a, w0_ref[...], preferred_element_type=jnp.float32)
    r = jnp.maximum(r + b0_ref[...], 0.0)
    r = jnp.dot(r.astype(jnp.bfloat16), w1_ref[...],
                preferred_element_type=jnp.float32)
    r = jnp.maximum(r + b1_ref[...], 0.0)
    o_ref[...] = jnp.dot(r.astype(jnp.bfloat16), w2_ref[...],
                         preferred_element_type=jnp.float32) + b2_ref[...]


def kernel(chunk_features, segment_ids, phi_w0, phi_b0, phi_w1, phi_b1,
           rho_w0, rho_b0, rho_w1, rho_b1, rho_w2, rho_b2):
    ids = segment_ids.astype(jnp.int32)
    pad = PAD_CHUNKS - N_CHUNKS
    ids = jnp.concatenate([ids, jnp.full((pad,), PAD_ID, jnp.int32)])
    ids3 = ids.reshape(NT, 1, T)

    agg = pl.pallas_call(
        _phi_seg_body,
        grid=(NT,),
        in_specs=[
            pl.BlockSpec((1, 1, T), lambda i: (i, 0, 0),
                         memory_space=pltpu.SMEM),
            pl.BlockSpec((T, IN_DIM), lambda i: (i, 0)),
            pl.BlockSpec((1, 1, T), lambda i: (i, 0, 0)),
            pl.BlockSpec((IN_DIM, HID), lambda i: (0, 0)),
            pl.BlockSpec((1, HID), lambda i: (0, 0)),
            pl.BlockSpec((HID, HID), lambda i: (0, 0)),
            pl.BlockSpec((1, HID), lambda i: (0, 0)),
        ],
        out_specs=pl.BlockSpec((PAD_CELLS, HID), lambda i: (0, 0)),
        out_shape=jax.ShapeDtypeStruct((PAD_CELLS, HID), jnp.float32),
        compiler_params=pltpu.CompilerParams(
            dimension_semantics=("arbitrary",)),
    )(ids3, chunk_features, ids3, phi_w0.astype(jnp.bfloat16),
      phi_b0.reshape(1, HID), phi_w1.astype(jnp.bfloat16),
      phi_b1.reshape(1, HID))

    out = pl.pallas_call(
        _rho_body,
        grid=(N_CELLS // CT,),
        in_specs=[
            pl.BlockSpec((CT, HID), lambda i: (i, 0)),
            pl.BlockSpec((HID, HID), lambda i: (0, 0)),
            pl.BlockSpec((1, HID), lambda i: (0, 0)),
            pl.BlockSpec((HID, HID), lambda i: (0, 0)),
            pl.BlockSpec((1, HID), lambda i: (0, 0)),
            pl.BlockSpec((HID, OUT_DIM), lambda i: (0, 0)),
            pl.BlockSpec((1, OUT_DIM), lambda i: (0, 0)),
        ],
        out_specs=pl.BlockSpec((CT, OUT_DIM), lambda i: (i, 0)),
        out_shape=jax.ShapeDtypeStruct((N_CELLS, OUT_DIM), jnp.float32),
    )(agg, rho_w0.astype(jnp.bfloat16), rho_b0.reshape(1, HID),
      rho_w1.astype(jnp.bfloat16), rho_b1.reshape(1, HID),
      rho_w2.astype(jnp.bfloat16), rho_b2.reshape(1, OUT_DIM))
    return out
